# baseline (speedup 1.0000x reference)
import numpy as np
import jax
import jax.numpy as jnp
from jax import lax
from jax.experimental import pallas as pl
from jax.experimental.pallas import tpu as pltpu
from jax.experimental.pallas import tpu_sc as plsc

_POOLING_COUNTS = [15, 8, 8]
_N = 64
_B = 32
_D = 512
_ROWS = _B * _D
_ROW_WORDS = _N * _N
_NW = 32
_RPW = _ROWS // _NW


def _build_static():
    mask2d = np.zeros((_N, _N), dtype=bool)
    mask2d[np.arange(_N), np.arange(_N)] = True
    entries = [(i, _N, i * _N + i) for i in range(_N)]
    stride, offset = 1, 0
    for c in _POOLING_COUNTS:
        for _ in range(c):
            offset += stride
            for i in range(0, _N - offset, stride):
                mask2d[i, i + offset] = True
                entries.append((i, i + offset, i * _N + i + offset))
        stride *= 2
    packed = np.array([(i << 19) | (j << 12) | p for i, j, p in entries],
                      dtype=np.int32)
    return mask2d, packed


_MASK2D_NP, _PACKED_NP = _build_static()
_NCHUNK = _PACKED_NP.size // 16


def _sc_body(x_hbm, tbl_hbm, out_hbm, xrow, tblv, rowbuf):
    cid = lax.axis_index("c")
    sid = lax.axis_index("s")
    wid = sid * 2 + cid
    base = wid * _RPW

    pltpu.sync_copy(tbl_hbm, tblv)

    zero = jnp.zeros((16,), jnp.float32)

    def _zero_body(k, carry):
        rowbuf[pl.ds(k * 16, 16)] = zero
        return carry

    lax.fori_loop(0, _ROW_WORDS // 16, _zero_body, 0)
    xrow[pl.ds(64, 16)] = zero

    def _row_body(r, carry):
        row = base + r
        pltpu.sync_copy(x_hbm.at[pl.ds(row * _N, _N)], xrow.at[pl.ds(0, _N)])
        for ch in range(_NCHUNK):
            p = tblv[pl.ds(ch * 16, 16)]
            iv = lax.shift_right_logical(p, 19)
            jv = lax.bitwise_and(lax.shift_right_logical(p, 12), 127)
            pos = lax.bitwise_and(p, 4095)
            xi = plsc.load_gather(xrow, [iv])
            xj = plsc.load_gather(xrow, [jv])
            plsc.store_scatter(rowbuf, [pos], xi + xj)
        pltpu.sync_copy(rowbuf, out_hbm.at[pl.ds(row * _ROW_WORDS, _ROW_WORDS)])
        return carry

    lax.fori_loop(0, _RPW, _row_body, 0)


_sc_call = pl.kernel(
    _sc_body,
    out_type=jax.ShapeDtypeStruct((_ROWS * _ROW_WORDS,), jnp.float32),
    mesh=plsc.VectorSubcoreMesh(core_axis_name="c", subcore_axis_name="s"),
    scratch_types=[
        pltpu.VMEM((80,), jnp.float32),
        pltpu.VMEM((_PACKED_NP.size,), jnp.int32),
        pltpu.VMEM((_ROW_WORDS,), jnp.float32),
    ],
    compiler_params=pltpu.CompilerParams(needs_layout_passes=False),
)


def kernel(x):
    B, D, N = x.shape
    x2 = x.reshape(B * D * N)
    tbl = jnp.asarray(_PACKED_NP)
    flat = _sc_call(x2, tbl)
    map2d = flat.reshape(B, D, N, N)
    mask2d = jnp.broadcast_to(
        jnp.asarray(_MASK2D_NP)[None, None, :, :], (B, 1, N, N))
    return (map2d, mask2d)


# speedup vs baseline: 1.4053x; 1.4053x over previous
import numpy as np
import jax
import jax.numpy as jnp
from jax import lax
from jax.experimental import pallas as pl
from jax.experimental.pallas import tpu as pltpu
from jax.experimental.pallas import tpu_sc as plsc

_POOLING_COUNTS = [15, 8, 8]
_N = 64
_B = 32
_D = 512
_ROWS = _B * _D
_RW = _N * _N
_NW = 32
_RPW = _ROWS // _NW
_RB = 4
_GW = _RB * _RW
_NG = _RPW // _RB
_XSTR = 80


def _build_static():
    mask2d = np.zeros((_N, _N), dtype=bool)
    mask2d[np.arange(_N), np.arange(_N)] = True
    entries = []
    stride, offset = 1, 0
    for c in _POOLING_COUNTS:
        for _ in range(c):
            offset += stride
            for i in range(0, _N - offset, stride):
                mask2d[i, i + offset] = True
                if not (offset <= 15 and i <= 48):
                    entries.append((i, i + offset, i * _N + i + offset))
        stride *= 2
    while len(entries) % 16:
        entries.append((_N, _N, 16))
    packed = np.array([(i << 19) | (j << 12) | p for i, j, p in entries],
                      dtype=np.int32)
    return mask2d, packed


_MASK2D_NP, _PACKED_NP = _build_static()
_NCHUNK = _PACKED_NP.size // 16


def _sc_body(x_hbm, tbl_hbm, out_hbm, xbuf, tblv, buf0, buf1, semx,
             sem0, sem1):
    cid = lax.axis_index("c")
    sid = lax.axis_index("s")
    wid = sid * 2 + cid
    base = wid * _RPW

    pltpu.sync_copy(tbl_hbm, tblv)

    zero = jnp.zeros((16,), jnp.float32)

    def _zero_body(k, carry):
        buf0[pl.ds(k * 16, 16)] = zero
        buf1[pl.ds(k * 16, 16)] = zero
        return carry

    lax.fori_loop(0, _GW // 16, _zero_body, 0)
    for rr in range(_RB):
        xbuf[pl.ds(rr * _XSTR + _N, 16)] = zero

    def _process(g, buf, sem):
        @pl.when(g >= 2)
        def _():
            pltpu.make_async_copy(buf, out_hbm.at[pl.ds(0, _GW)], sem).wait()

        descs = []
        for rr in range(_RB):
            row = base + g * _RB + rr
            descs.append(pltpu.async_copy(
                x_hbm.at[pl.ds(row * _N, _N)],
                xbuf.at[pl.ds(rr * _XSTR, _N)], semx))
        for d in descs:
            d.wait()

        for rr in range(_RB):
            xb = rr * _XSTR
            ob = rr * _RW
            for i in range(49):
                xw = xbuf[pl.ds(xb + i, 16)]
                buf[pl.ds(ob + i * 65, 16)] = xw + xw[0]
            for ch in range(_NCHUNK):
                p = tblv[pl.ds(ch * 16, 16)]
                iv = lax.shift_right_logical(p, 19) + xb
                jv = lax.bitwise_and(lax.shift_right_logical(p, 12), 127) + xb
                pos = lax.bitwise_and(p, 4095) + ob
                xi = plsc.load_gather(xbuf, [iv])
                xj = plsc.load_gather(xbuf, [jv])
                plsc.store_scatter(buf, [pos], xi + xj)
            for c in range(4):
                xv = xbuf[pl.ds(xb + 16 * c, 16)]
                pos = lax.iota(jnp.int32, 16) * 65 + (ob + c * 16 * 65)
                plsc.store_scatter(buf, [pos], xv)

        pltpu.async_copy(buf, out_hbm.at[pl.ds((base + g * _RB) * _RW, _GW)],
                         sem)

    def _pair(i, carry):
        _process(2 * i, buf0, sem0)
        _process(2 * i + 1, buf1, sem1)
        return carry

    lax.fori_loop(0, _NG // 2, _pair, 0)
    pltpu.make_async_copy(buf0, out_hbm.at[pl.ds(0, _GW)], sem0).wait()
    pltpu.make_async_copy(buf1, out_hbm.at[pl.ds(0, _GW)], sem1).wait()


_sc_call = pl.kernel(
    _sc_body,
    out_type=jax.ShapeDtypeStruct((_ROWS * _RW,), jnp.float32),
    mesh=plsc.VectorSubcoreMesh(core_axis_name="c", subcore_axis_name="s"),
    scratch_types=[
        pltpu.VMEM((_RB * _XSTR,), jnp.float32),
        pltpu.VMEM((_PACKED_NP.size,), jnp.int32),
        pltpu.VMEM((_GW,), jnp.float32),
        pltpu.VMEM((_GW,), jnp.float32),
        pltpu.SemaphoreType.DMA,
        pltpu.SemaphoreType.DMA,
        pltpu.SemaphoreType.DMA,
    ],
    compiler_params=pltpu.CompilerParams(needs_layout_passes=False),
)


def kernel(x):
    B, D, N = x.shape
    x2 = x.reshape(B * D * N)
    tbl = jnp.asarray(_PACKED_NP)
    flat = _sc_call(x2, tbl)
    map2d = flat.reshape(B, D, N, N)
    mask2d = jnp.broadcast_to(
        jnp.asarray(_MASK2D_NP)[None, None, :, :], (B, 1, N, N))
    return (map2d, mask2d)


# speedup vs baseline: 1.8397x; 1.3091x over previous
import numpy as np
import jax
import jax.numpy as jnp
from jax import lax
from jax.experimental import pallas as pl
from jax.experimental.pallas import tpu as pltpu
from jax.experimental.pallas import tpu_sc as plsc

_POOLING_COUNTS = [15, 8, 8]
_N = 64
_B = 32
_D = 512
_ROWS = _B * _D
_RW = _N * _N
_NW = 32
_RPW = _ROWS // _NW
_RB = 4
_GW = _RB * _RW
_NG = _RPW // _RB
_XW = _RPW * _N
_ZOFF = _XW


def _build_static():
    mask2d = np.zeros((_N, _N), dtype=bool)
    mask2d[np.arange(_N), np.arange(_N)] = True
    pairs = []
    stride, offset = 1, 0
    for c in _POOLING_COUNTS:
        for _ in range(c):
            offset += stride
            for i in range(0, _N - offset, stride):
                mask2d[i, i + offset] = True
                if offset > 15:
                    pairs.append((i, i + offset))
        stride *= 2
    entries = []
    for rr in range(_RB):
        for i, j in pairs:
            entries.append(((rr * _N + i) << 23
                            | (rr * _N + j) << 14
                            | (rr * _RW + i * _N + j)))
    assert len(entries) % 16 == 0
    packed = np.array(entries, dtype=np.uint32).view(np.int32)
    return mask2d, packed


_MASK2D_NP, _PACKED_NP = _build_static()
_NCHUNK = _PACKED_NP.size // 16


def _sc_body(x_hbm, tbl_hbm, out_hbm, xtile, tblv, buf0, buf1, sem0, sem1):
    cid = lax.axis_index("c")
    sid = lax.axis_index("s")
    wid = sid * 2 + cid
    base = wid * _RPW

    pltpu.sync_copy(tbl_hbm, tblv)
    pltpu.sync_copy(x_hbm.at[pl.ds(base * _N, _XW)], xtile.at[pl.ds(0, _XW)])

    zero = jnp.zeros((16,), jnp.float32)
    xtile[pl.ds(_ZOFF, 16)] = zero

    @plsc.parallel_loop(0, _GW // 16)
    def _zero_body(k):
        buf0[pl.ds(k * 16, 16)] = zero
        buf1[pl.ds(k * 16, 16)] = zero

    iota = lax.iota(jnp.int32, 16)
    iota65 = iota * 65

    def _process(g, buf, sem):
        @pl.when(g >= 2)
        def _():
            pltpu.make_async_copy(buf, out_hbm.at[pl.ds(0, _GW)], sem).wait()

        xg = g * (_RB * _N)

        @plsc.parallel_loop(0, _RB * 64, unroll=8)
        def _runs(u):
            rr = lax.shift_right_logical(u, 6)
            c = lax.bitwise_and(lax.shift_right_logical(u, 4), 3)
            l = lax.bitwise_and(u, 15)
            xoff = xg + rr * _N + c * 16
            xc = xtile[pl.ds(xoff, 16)]
            xl = xtile[pl.ds(jnp.where(l == 0, _ZOFF, xoff + l), 16)]
            pos = iota65 + (rr * _RW + c * (16 * 65) + l)
            mask = iota < (_N - c * 16 - l)
            plsc.store_scatter(buf, [pos], xc + xl, mask=mask)

        @plsc.parallel_loop(0, _NCHUNK, unroll=4)
        def _strided(v):
            p = tblv[pl.ds(v * 16, 16)]
            iv = lax.shift_right_logical(p, 23) + xg
            jv = lax.bitwise_and(lax.shift_right_logical(p, 14), 511) + xg
            pos = lax.bitwise_and(p, 16383)
            xi = plsc.load_gather(xtile, [iv])
            xj = plsc.load_gather(xtile, [jv])
            plsc.store_scatter(buf, [pos], xi + xj)

        pltpu.async_copy(buf, out_hbm.at[pl.ds((base + g * _RB) * _RW, _GW)],
                         sem)

    def _pair(i, carry):
        _process(2 * i, buf0, sem0)
        _process(2 * i + 1, buf1, sem1)
        return carry

    lax.fori_loop(0, _NG // 2, _pair, 0)
    pltpu.make_async_copy(buf0, out_hbm.at[pl.ds(0, _GW)], sem0).wait()
    pltpu.make_async_copy(buf1, out_hbm.at[pl.ds(0, _GW)], sem1).wait()


_sc_call = pl.kernel(
    _sc_body,
    out_type=jax.ShapeDtypeStruct((_ROWS * _RW,), jnp.float32),
    mesh=plsc.VectorSubcoreMesh(core_axis_name="c", subcore_axis_name="s"),
    scratch_types=[
        pltpu.VMEM((_XW + 16,), jnp.float32),
        pltpu.VMEM((_PACKED_NP.size,), jnp.int32),
        pltpu.VMEM((_GW,), jnp.float32),
        pltpu.VMEM((_GW,), jnp.float32),
        pltpu.SemaphoreType.DMA,
        pltpu.SemaphoreType.DMA,
    ],
    compiler_params=pltpu.CompilerParams(needs_layout_passes=False),
)


def kernel(x):
    B, D, N = x.shape
    x2 = x.reshape(B * D * N)
    tbl = jnp.asarray(_PACKED_NP)
    flat = _sc_call(x2, tbl)
    map2d = flat.reshape(B, D, N, N)
    mask2d = jnp.broadcast_to(
        jnp.asarray(_MASK2D_NP)[None, None, :, :], (B, 1, N, N))
    return (map2d, mask2d)


# speedup vs baseline: 3.1849x; 1.7312x over previous
import numpy as np
import jax
import jax.numpy as jnp
from jax import lax
from jax.experimental import pallas as pl
from jax.experimental.pallas import tpu as pltpu
from jax.experimental.pallas import tpu_sc as plsc

_POOLING_COUNTS = [15, 8, 8]
_N = 64
_B = 32
_D = 512
_ROWS = _B * _D
_RW = _N * _N
_NW = 32
_RPW = _ROWS // _NW
_RB = 4
_GW = _RB * _RW
_NG = _RPW // _RB
_XW = _RPW * _N
_ZOFF = _XW


def _build_static():
    mask2d = np.zeros((_N, _N), dtype=bool)
    mask2d[np.arange(_N), np.arange(_N)] = True
    pairs = []
    stride, offset = 1, 0
    for c in _POOLING_COUNTS:
        for _ in range(c):
            offset += stride
            for i in range(0, _N - offset, stride):
                mask2d[i, i + offset] = True
                if offset > 15:
                    pairs.append((i, i + offset))
        stride *= 2
    entries = []
    for rr in range(_RB):
        for i, j in pairs:
            entries.append(((rr * _N + i) << 6) | j)
    assert len(entries) % 16 == 0
    packed = np.array(entries, dtype=np.int32)
    return mask2d, packed


_MASK2D_NP, _PACKED_NP = _build_static()
_NCHUNK = _PACKED_NP.size // 16


def _sc_body(x_hbm, tbl_hbm, out_hbm, xtile, tblv, buf0, buf1, sem0, sem1):
    cid = lax.axis_index("c")
    sid = lax.axis_index("s")
    wid = sid * 2 + cid
    base = wid * _RPW

    pltpu.sync_copy(tbl_hbm, tblv)
    pltpu.sync_copy(x_hbm.at[pl.ds(base * _N, _XW)], xtile.at[pl.ds(0, _XW)])

    zero = jnp.zeros((16,), jnp.float32)
    xtile[pl.ds(_ZOFF, 16)] = zero

    @plsc.parallel_loop(0, _RB * _N * (_N // 16))
    def _zero_body(k):
        rr = lax.shift_right_logical(k, 8)
        i = lax.bitwise_and(lax.shift_right_logical(k, 2), 63)
        q = lax.bitwise_and(k, 3)
        buf0[rr, i, pl.ds(q * 16, 16)] = zero
        buf1[rr, i, pl.ds(q * 16, 16)] = zero

    iota = lax.iota(jnp.int32, 16)

    def _process(g, buf, sem):
        @pl.when(g >= 2)
        def _():
            pltpu.make_async_copy(buf, out_hbm.at[pl.ds(0, _RB)], sem).wait()

        xg = g * (_RB * _N)

        @plsc.parallel_loop(0, _RB * 64, unroll=8)
        def _runs(u):
            rr = lax.shift_right_logical(u, 6)
            c = lax.bitwise_and(lax.shift_right_logical(u, 4), 3)
            l = lax.bitwise_and(u, 15)
            xoff = xg + rr * _N + c * 16
            xc = xtile[pl.ds(xoff, 16)]
            xl = xtile[pl.ds(jnp.where(l == 0, _ZOFF, xoff + l), 16)]
            ivec = iota + c * 16
            rvec = jnp.broadcast_to(rr, (16,))
            mask = iota < (_N - c * 16 - l)
            plsc.store_scatter(buf, [rvec, ivec, ivec + l], xc + xl,
                               mask=mask)

        @plsc.parallel_loop(0, _NCHUNK, unroll=4)
        def _strided(v):
            p = tblv[pl.ds(v * 16, 16)]
            xidx = lax.shift_right_logical(p, 6)
            jvec = lax.bitwise_and(p, 63)
            rvec = lax.shift_right_logical(xidx, 6)
            ivec = lax.bitwise_and(xidx, 63)
            xi = plsc.load_gather(xtile, [xidx + xg])
            xj = plsc.load_gather(xtile, [(xidx - ivec) + (jvec + xg)])
            plsc.store_scatter(buf, [rvec, ivec, jvec], xi + xj)

        pltpu.async_copy(buf, out_hbm.at[pl.ds(base + g * _RB, _RB)], sem)

    def _pair(i, carry):
        _process(2 * i, buf0, sem0)
        _process(2 * i + 1, buf1, sem1)
        return carry

    lax.fori_loop(0, _NG // 2, _pair, 0)
    pltpu.make_async_copy(buf0, out_hbm.at[pl.ds(0, _RB)], sem0).wait()
    pltpu.make_async_copy(buf1, out_hbm.at[pl.ds(0, _RB)], sem1).wait()


_sc_call = pl.kernel(
    _sc_body,
    out_type=jax.ShapeDtypeStruct((_ROWS, _N, _N), jnp.float32),
    mesh=plsc.VectorSubcoreMesh(core_axis_name="c", subcore_axis_name="s"),
    scratch_types=[
        pltpu.VMEM((_XW + 16,), jnp.float32),
        pltpu.VMEM((_PACKED_NP.size,), jnp.int32),
        pltpu.VMEM((_RB, _N, _N), jnp.float32),
        pltpu.VMEM((_RB, _N, _N), jnp.float32),
        pltpu.SemaphoreType.DMA,
        pltpu.SemaphoreType.DMA,
    ],
    compiler_params=pltpu.CompilerParams(needs_layout_passes=False),
)


def kernel(x):
    B, D, N = x.shape
    x2 = x.reshape(B * D * N)
    tbl = jnp.asarray(_PACKED_NP)
    out3 = _sc_call(x2, tbl)
    map2d = out3.reshape(B, D, N, N)
    mask2d = jnp.broadcast_to(
        jnp.asarray(_MASK2D_NP)[None, None, :, :], (B, 1, N, N))
    return (map2d, mask2d)


# speedup vs baseline: 10.7194x; 3.3658x over previous
import numpy as np
import jax
import jax.numpy as jnp
from jax import lax
from jax.experimental import pallas as pl
from jax.experimental.pallas import tpu as pltpu
from jax.experimental.pallas import tpu_sc as plsc

_POOLING_COUNTS = [15, 8, 8]
_N = 64
_B = 32
_D = 512
_NW = 32
_NQ = _D // 16
_XW = _N * _D


def _active_sets():
    mask2d = np.zeros((_N, _N), dtype=bool)
    mask2d[np.arange(_N), np.arange(_N)] = True
    stride, offset = 1, 0
    for c in _POOLING_COUNTS:
        for _ in range(c):
            offset += stride
            i = np.arange(0, _N - offset, stride)
            mask2d[i, i + offset] = True
        stride *= 2
    return mask2d


def _build_static():
    mask2d = _active_sets()
    entries, starts = [], [0]
    for i in range(_N):
        act = [j for j in range(_N) if mask2d[i, j]]
        prev = [j for j in range(_N) if i >= 2 and mask2d[i - 2, j]]
        stale = [j for j in prev if not mask2d[i, j]]
        entries.extend(act)
        entries.extend(j | 64 for j in stale)
        starts.append(len(entries))
    entries.extend([64 + 63] * 16)
    starts.extend([starts[-1]] * 15)
    return (mask2d,
            np.array(entries, dtype=np.int32),
            np.array(starts, dtype=np.int32))


_MASK2D_NP, _ENTRIES_NP, _STARTS_NP = _build_static()


def _sc_body(x_hbm, tbl_hbm, st_hbm, out_hbm, xtile, tblv, stv, buf0, buf1,
             sem0, sem1):
    cid = lax.axis_index("c")
    sid = lax.axis_index("s")
    wid = sid * 2 + cid

    pltpu.sync_copy(tbl_hbm, tblv)
    pltpu.sync_copy(st_hbm, stv)
    pltpu.sync_copy(x_hbm.at[pl.ds(wid * _XW, _XW)], xtile.at[pl.ds(0, _XW)])

    zero = jnp.zeros((16,), jnp.float32)

    @plsc.parallel_loop(0, _XW // 16)
    def _zero_body(k):
        j = lax.shift_right_logical(k, 5)
        q = lax.bitwise_and(k, 31)
        buf0[j, pl.ds(q * 16, 16)] = zero
        buf1[j, pl.ds(q * 16, 16)] = zero

    def _process(g, buf, sem):
        @pl.when(g >= 2)
        def _():
            pltpu.make_async_copy(buf, out_hbm.at[0, 0], sem).wait()

        sv = stv[pl.ds(g, 16)]
        e0 = sv[0]
        e1 = sv[1]
        goff = g * _D

        @plsc.parallel_loop(e0, e1)
        def _jobs(e):
            ev = tblv[pl.ds(e, 16)]
            ent = ev[0]
            j = lax.bitwise_and(ent, 63)
            keep = (1 - lax.shift_right_logical(ent, 6)).astype(jnp.float32)
            kv = jnp.broadcast_to(keep, (16,))
            joff = j * _D
            for q in range(_NQ):
                xi = xtile[pl.ds(goff + q * 16, 16)]
                xj = xtile[pl.ds(joff + q * 16, 16)]
                buf[j, pl.ds(q * 16, 16)] = (xi + xj) * kv

        pltpu.async_copy(buf, out_hbm.at[wid, g], sem)

    def _pair(t, carry):
        _process(2 * t, buf0, sem0)
        _process(2 * t + 1, buf1, sem1)
        return carry

    lax.fori_loop(0, _N // 2, _pair, 0)
    pltpu.make_async_copy(buf0, out_hbm.at[0, 0], sem0).wait()
    pltpu.make_async_copy(buf1, out_hbm.at[0, 0], sem1).wait()


_sc_call = pl.kernel(
    _sc_body,
    out_type=jax.ShapeDtypeStruct((_B, _N, _N, _D), jnp.float32),
    mesh=plsc.VectorSubcoreMesh(core_axis_name="c", subcore_axis_name="s"),
    scratch_types=[
        pltpu.VMEM((_XW,), jnp.float32),
        pltpu.VMEM((_ENTRIES_NP.size,), jnp.int32),
        pltpu.VMEM((_STARTS_NP.size,), jnp.int32),
        pltpu.VMEM((_N, _D), jnp.float32),
        pltpu.VMEM((_N, _D), jnp.float32),
        pltpu.SemaphoreType.DMA,
        pltpu.SemaphoreType.DMA,
    ],
    compiler_params=pltpu.CompilerParams(needs_layout_passes=False),
)


def kernel(x):
    B, D, N = x.shape
    xt = jnp.swapaxes(x, 1, 2).reshape(B * N * D)
    out4 = _sc_call(xt, jnp.asarray(_ENTRIES_NP), jnp.asarray(_STARTS_NP))
    map2d = jnp.transpose(out4, (0, 3, 1, 2))
    mask2d = jnp.broadcast_to(
        jnp.asarray(_MASK2D_NP)[None, None, :, :], (B, 1, N, N))
    return (map2d, mask2d)
